# baseline (speedup 1.0000x reference)
import jax
import jax.numpy as jnp
from jax.experimental import pallas as pl
from jax.experimental.pallas import tpu as pltpu

_VMEM = 100 * 1024 * 1024


def _shift_w(xp, d):
    if d == 0:
        return xp
    S = xp.shape[2]
    if d == 1:
        p0 = xp[:, :, 0]
        p0 = jnp.concatenate([p0[:, :, 1:], jnp.zeros_like(p0[:, :, :1])],
                             axis=2)
        return jnp.concatenate([xp[:, :, 1:], p0[:, :, None]], axis=2)
    pl_ = xp[:, :, S - 1]
    pl_ = jnp.concatenate([jnp.zeros_like(pl_[:, :, :1]), pl_[:, :, :-1]],
                          axis=2)
    return jnp.concatenate([pl_[:, :, None], xp[:, :, :S - 1]], axis=2)


def _conv3x3(x, w9, b, pool, p_ref):
    BT, H, S, V, C = x.shape
    M = BT * H * S * V
    Cout = w9.shape[1]
    p_ref[:, 0] = jnp.zeros((BT, S, V, 3 * C), jnp.bfloat16)
    p_ref[:, H + 1] = jnp.zeros((BT, S, V, 3 * C), jnp.bfloat16)
    for d in (-1, 0, 1):
        p_ref[:, 1:H + 1, :, :, (d + 1) * C:(d + 2) * C] = _shift_w(x, d)
    y = sum(jnp.dot(p_ref[:, dy:dy + H].reshape(M, 3 * C),
                    w9[3 * C * dy:3 * C * (dy + 1)],
                    preferred_element_type=jnp.float32)
            for dy in range(3))
    y = jnp.maximum(y + b, 0.0).astype(jnp.bfloat16)
    if pool:
        y = y.reshape(BT, H, S // 2, 2, V, Cout)
        y = jnp.maximum(y[:, :, :, 0], y[:, :, :, 1])
        y = y.reshape(BT, H // 2, 2, S // 2, V, Cout)
        y = jnp.maximum(y[:, :, 0], y[:, :, 1])
        return y
    return y.reshape(BT, H, S, V, Cout)


def _conv1_nhwc(x_ref, w9, b, p_ref):
    xt = jnp.swapaxes(x_ref[...], -1, -2)
    x = xt.astype(jnp.bfloat16)
    BT, H, W, C = x.shape
    M = BT * H * W
    z = jnp.zeros((BT, H, 1, C), x.dtype)
    xp = jnp.concatenate([z, x, z], axis=2)
    p_ref[:, 0] = jnp.zeros((BT, W, 3 * C), jnp.bfloat16)
    p_ref[:, H + 1] = jnp.zeros((BT, W, 3 * C), jnp.bfloat16)
    for dx in range(3):
        p_ref[:, 1:H + 1, :, dx * C:(dx + 1) * C] = xp[:, :, dx:dx + W, :]
    y = sum(jnp.dot(p_ref[:, dy:dy + H].reshape(M, 3 * C),
                    w9[3 * C * dy:3 * C * (dy + 1)],
                    preferred_element_type=jnp.float32)
            for dy in range(3))
    y = jnp.maximum(y + b, 0.0).astype(jnp.bfloat16)
    y = y.reshape(BT, H, 8, 8, 32)
    return jnp.transpose(y, (0, 1, 3, 2, 4))


def _trunk_body(x_ref, w0, b0, w1, b1, w2, b2, w3, b3, w4, b4, w5, b5, o_ref,
                p1, p2, p3, p4, p5, p6):
    y = _conv1_nhwc(x_ref, w0[...], b0[...], p1)
    y = _conv3x3(y, w1[...], b1[...], True, p2)
    y = _conv3x3(y, w2[...], b2[...], False, p3)
    y = _conv3x3(y, w3[...], b3[...], True, p4)
    y = _conv3x3(y, w4[...], b4[...], False, p5)
    y = _conv3x3(y, w5[...], b5[...], True, p6)
    o_ref[...] = y.reshape(y.shape[0], 64, 128)


def _head_body(f_ref, w1_ref, b1_ref, w2_ref, b2_ref, o_ref):
    h = jnp.dot(f_ref[...], w1_ref[...], preferred_element_type=jnp.float32)
    h = jnp.maximum(h + b1_ref[...], 0.0)
    o_ref[...] = (jnp.dot(h, w2_ref[...], preferred_element_type=jnp.float32)
                  + b2_ref[...])


def kernel(x, cw0, cb0, cw1, cb1, cw2, cb2, cw3, cb3, cw4, cb4, cw5, cb5,
           fw0, fb0, fw1, fb1):
    B = x.shape[0]
    BT = 2
    xh = jnp.transpose(x, (0, 2, 1, 3))

    ws, bs = [], []
    for w, b in ((cw0, cb0), (cw1, cb1), (cw2, cb2),
                 (cw3, cb3), (cw4, cb4), (cw5, cb5)):
        cin, cout = w.shape[2], w.shape[3]
        ws.append(w.reshape(9 * cin, cout).astype(jnp.bfloat16))
        bs.append(b.reshape(1, cout))

    def _whole(shape):
        return pl.BlockSpec(shape, lambda i: (0,) * len(shape))

    in_specs = [pl.BlockSpec((BT, 64, 3, 64), lambda i: (i, 0, 0, 0))]
    args = [xh]
    for w9, b in zip(ws, bs):
        in_specs += [_whole(w9.shape), _whole(b.shape)]
        args += [w9, b]

    flat3 = pl.pallas_call(
        _trunk_body,
        out_shape=jax.ShapeDtypeStruct((B, 64, 128), jnp.bfloat16),
        grid=(B // BT,),
        in_specs=in_specs,
        out_specs=pl.BlockSpec((BT, 64, 128), lambda i: (i, 0, 0)),
        scratch_shapes=[
            pltpu.VMEM((BT, 66, 64, 9), jnp.bfloat16),
            pltpu.VMEM((BT, 66, 8, 8, 96), jnp.bfloat16),
            pltpu.VMEM((BT, 34, 4, 8, 96), jnp.bfloat16),
            pltpu.VMEM((BT, 34, 4, 8, 192), jnp.bfloat16),
            pltpu.VMEM((BT, 18, 2, 8, 192), jnp.bfloat16),
            pltpu.VMEM((BT, 18, 2, 8, 384), jnp.bfloat16),
        ],
        compiler_params=pltpu.CompilerParams(
            dimension_semantics=("parallel",), vmem_limit_bytes=_VMEM),
    )(*args)
    flat = flat3.reshape(B, 64 * 128)

    fw0p = (fw0.reshape(128, 8, 8, 512).transpose(1, 2, 0, 3)
            .reshape(8192, 512))

    TM = min(256, B)
    out = pl.pallas_call(
        _head_body,
        out_shape=jax.ShapeDtypeStruct((B, 10), jnp.float32),
        grid=(B // TM,),
        in_specs=[pl.BlockSpec((TM, 8192), lambda i: (i, 0)),
                  _whole((8192, 512)), _whole((1, 512)),
                  _whole((512, 10)), _whole((1, 10))],
        out_specs=pl.BlockSpec((TM, 10), lambda i: (i, 0)),
        compiler_params=pltpu.CompilerParams(
            dimension_semantics=("parallel",), vmem_limit_bytes=_VMEM),
    )(flat, fw0p, fb0.reshape(1, 512), fw1, fb1.reshape(1, 10))
    return out


# speedup vs baseline: 1.0365x; 1.0365x over previous
import jax
import jax.numpy as jnp
from jax.experimental import pallas as pl
from jax.experimental.pallas import tpu as pltpu

_VMEM = 100 * 1024 * 1024


def _shift_w(xp, d):
    if d == 0:
        return xp
    S = xp.shape[2]
    if d == 1:
        p0 = xp[:, :, 0]
        p0 = jnp.concatenate([p0[:, :, 1:], jnp.zeros_like(p0[:, :, :1])],
                             axis=2)
        return jnp.concatenate([xp[:, :, 1:], p0[:, :, None]], axis=2)
    pl_ = xp[:, :, S - 1]
    pl_ = jnp.concatenate([jnp.zeros_like(pl_[:, :, :1]), pl_[:, :, :-1]],
                          axis=2)
    return jnp.concatenate([pl_[:, :, None], xp[:, :, :S - 1]], axis=2)


def _conv3x3(x, w9, b, pool, p_ref):
    BT, H, S, V, C = x.shape
    M = BT * H * S * V
    Cout = w9.shape[1]
    p_ref[:, 0] = jnp.zeros((BT, S, V, 3 * C), jnp.bfloat16)
    p_ref[:, H + 1] = jnp.zeros((BT, S, V, 3 * C), jnp.bfloat16)
    for d in (-1, 0, 1):
        p_ref[:, 1:H + 1, :, :, (d + 1) * C:(d + 2) * C] = _shift_w(x, d)
    y = sum(jnp.dot(p_ref[:, dy:dy + H].reshape(M, 3 * C),
                    w9[3 * C * dy:3 * C * (dy + 1)],
                    preferred_element_type=jnp.float32)
            for dy in range(3))
    y = jnp.maximum(y + b, 0.0).astype(jnp.bfloat16)
    if pool:
        y = y.reshape(BT, H, S // 2, 2, V, Cout)
        y = jnp.maximum(y[:, :, :, 0], y[:, :, :, 1])
        y = y.reshape(BT, H // 2, 2, S // 2, V, Cout)
        y = jnp.maximum(y[:, :, 0], y[:, :, 1])
        return y
    return y.reshape(BT, H, S, V, Cout)


def _conv1_nhwc(x_ref, w9, b, p_ref):
    xt = jnp.swapaxes(x_ref[...], -1, -2)
    x = xt.astype(jnp.bfloat16)
    BT, H, W, C = x.shape
    M = BT * H * W
    z = jnp.zeros((BT, H, 1, C), x.dtype)
    xp = jnp.concatenate([z, x, z], axis=2)
    p_ref[:, 0] = jnp.zeros((BT, W, 3 * C), jnp.bfloat16)
    p_ref[:, H + 1] = jnp.zeros((BT, W, 3 * C), jnp.bfloat16)
    for dx in range(3):
        p_ref[:, 1:H + 1, :, dx * C:(dx + 1) * C] = xp[:, :, dx:dx + W, :]
    y = sum(jnp.dot(p_ref[:, dy:dy + H].reshape(M, 3 * C),
                    w9[3 * C * dy:3 * C * (dy + 1)],
                    preferred_element_type=jnp.float32)
            for dy in range(3))
    y = jnp.maximum(y + b, 0.0).astype(jnp.bfloat16)
    y = y.reshape(BT, H, 8, 8, 32)
    return jnp.transpose(y, (0, 1, 3, 2, 4))


def _trunk_body(x_ref, w0, b0, w1, b1, w2, b2, w3, b3, w4, b4, w5, b5, o_ref,
                p1, p2, p3, p4, p5, p6):
    y = _conv1_nhwc(x_ref, w0[...], b0[...], p1)
    y = _conv3x3(y, w1[...], b1[...], True, p2)
    y = _conv3x3(y, w2[...], b2[...], False, p3)
    y = _conv3x3(y, w3[...], b3[...], True, p4)
    y = _conv3x3(y, w4[...], b4[...], False, p5)
    y = _conv3x3(y, w5[...], b5[...], True, p6)
    o_ref[...] = y.reshape(y.shape[0], 64, 128)


def _head_body(f_ref, w1_ref, b1_ref, w2_ref, b2_ref, o_ref):
    h = jnp.dot(f_ref[...], w1_ref[...], preferred_element_type=jnp.float32)
    h = jnp.maximum(h + b1_ref[...], 0.0)
    o_ref[...] = (jnp.dot(h, w2_ref[...], preferred_element_type=jnp.float32)
                  + b2_ref[...])


def kernel(x, cw0, cb0, cw1, cb1, cw2, cb2, cw3, cb3, cw4, cb4, cw5, cb5,
           fw0, fb0, fw1, fb1):
    B = x.shape[0]
    BT = 4
    xh = jnp.transpose(x, (0, 2, 1, 3))

    ws, bs = [], []
    for w, b in ((cw0, cb0), (cw1, cb1), (cw2, cb2),
                 (cw3, cb3), (cw4, cb4), (cw5, cb5)):
        cin, cout = w.shape[2], w.shape[3]
        ws.append(w.reshape(9 * cin, cout).astype(jnp.bfloat16))
        bs.append(b.reshape(1, cout))

    def _whole(shape):
        return pl.BlockSpec(shape, lambda i: (0,) * len(shape))

    in_specs = [pl.BlockSpec((BT, 64, 3, 64), lambda i: (i, 0, 0, 0))]
    args = [xh]
    for w9, b in zip(ws, bs):
        in_specs += [_whole(w9.shape), _whole(b.shape)]
        args += [w9, b]

    flat3 = pl.pallas_call(
        _trunk_body,
        out_shape=jax.ShapeDtypeStruct((B, 64, 128), jnp.bfloat16),
        grid=(B // BT,),
        in_specs=in_specs,
        out_specs=pl.BlockSpec((BT, 64, 128), lambda i: (i, 0, 0)),
        scratch_shapes=[
            pltpu.VMEM((BT, 66, 64, 9), jnp.bfloat16),
            pltpu.VMEM((BT, 66, 8, 8, 96), jnp.bfloat16),
            pltpu.VMEM((BT, 34, 4, 8, 96), jnp.bfloat16),
            pltpu.VMEM((BT, 34, 4, 8, 192), jnp.bfloat16),
            pltpu.VMEM((BT, 18, 2, 8, 192), jnp.bfloat16),
            pltpu.VMEM((BT, 18, 2, 8, 384), jnp.bfloat16),
        ],
        compiler_params=pltpu.CompilerParams(
            dimension_semantics=("parallel",), vmem_limit_bytes=_VMEM),
    )(*args)
    flat = flat3.reshape(B, 64 * 128)

    fw0p = (fw0.reshape(128, 8, 8, 512).transpose(1, 2, 0, 3)
            .reshape(8192, 512))

    TM = min(256, B)
    out = pl.pallas_call(
        _head_body,
        out_shape=jax.ShapeDtypeStruct((B, 10), jnp.float32),
        grid=(B // TM,),
        in_specs=[pl.BlockSpec((TM, 8192), lambda i: (i, 0)),
                  _whole((8192, 512)), _whole((1, 512)),
                  _whole((512, 10)), _whole((1, 10))],
        out_specs=pl.BlockSpec((TM, 10), lambda i: (i, 0)),
        compiler_params=pltpu.CompilerParams(
            dimension_semantics=("parallel",), vmem_limit_bytes=_VMEM),
    )(flat, fw0p, fb0.reshape(1, 512), fw1, fb1.reshape(1, 10))
    return out


# speedup vs baseline: 1.0514x; 1.0143x over previous
import jax
import jax.numpy as jnp
from jax.experimental import pallas as pl
from jax.experimental.pallas import tpu as pltpu

_VMEM = 100 * 1024 * 1024


def _shift_w(xp, d):
    if d == 0:
        return xp
    S = xp.shape[2]
    if d == 1:
        p0 = xp[:, :, 0]
        p0 = jnp.concatenate([p0[:, :, 1:], jnp.zeros_like(p0[:, :, :1])],
                             axis=2)
        return jnp.concatenate([xp[:, :, 1:], p0[:, :, None]], axis=2)
    pl_ = xp[:, :, S - 1]
    pl_ = jnp.concatenate([jnp.zeros_like(pl_[:, :, :1]), pl_[:, :, :-1]],
                          axis=2)
    return jnp.concatenate([pl_[:, :, None], xp[:, :, :S - 1]], axis=2)


def _conv3x3(x, w9, b, pool, p_ref):
    BT, H, S, V, C = x.shape
    M = BT * H * S * V
    Cout = w9.shape[1]
    p_ref[:, 0] = jnp.zeros((BT, S, V, 3 * C), jnp.bfloat16)
    p_ref[:, H + 1] = jnp.zeros((BT, S, V, 3 * C), jnp.bfloat16)
    for d in (-1, 0, 1):
        p_ref[:, 1:H + 1, :, :, (d + 1) * C:(d + 2) * C] = _shift_w(x, d)
    y = sum(jnp.dot(p_ref[:, dy:dy + H].reshape(M, 3 * C),
                    w9[3 * C * dy:3 * C * (dy + 1)],
                    preferred_element_type=jnp.float32)
            for dy in range(3))
    y = jnp.maximum(y + b, 0.0).astype(jnp.bfloat16)
    if pool:
        y = y.reshape(BT, H, S // 2, 2, V, Cout)
        y = jnp.maximum(y[:, :, :, 0], y[:, :, :, 1])
        y = y.reshape(BT, H // 2, 2, S // 2, V, Cout)
        y = jnp.maximum(y[:, :, 0], y[:, :, 1])
        return y
    return y.reshape(BT, H, S, V, Cout)


def _conv1_nhwc(x_ref, w9, b, p_ref):
    xt = jnp.swapaxes(x_ref[...].astype(jnp.bfloat16), -1, -2)
    x = xt
    BT, H, W, C = x.shape
    M = BT * H * W
    z = jnp.zeros((BT, H, 1, C), x.dtype)
    xp = jnp.concatenate([z, x, z], axis=2)
    p_ref[:, 0] = jnp.zeros((BT, W, 3 * C), jnp.bfloat16)
    p_ref[:, H + 1] = jnp.zeros((BT, W, 3 * C), jnp.bfloat16)
    for dx in range(3):
        p_ref[:, 1:H + 1, :, dx * C:(dx + 1) * C] = xp[:, :, dx:dx + W, :]
    y = sum(jnp.dot(p_ref[:, dy:dy + H].reshape(M, 3 * C),
                    w9[3 * C * dy:3 * C * (dy + 1)],
                    preferred_element_type=jnp.float32)
            for dy in range(3))
    y = jnp.maximum(y + b, 0.0).astype(jnp.bfloat16)
    y = y.reshape(BT, H, 8, 8, 32)
    return jnp.transpose(y, (0, 1, 3, 2, 4))


def _trunk_body(x_ref, w0, b0, w1, b1, w2, b2, w3, b3, w4, b4, w5, b5, o_ref,
                p1, p2, p3, p4, p5, p6):
    y = _conv1_nhwc(x_ref, w0[...], b0[...], p1)
    y = _conv3x3(y, w1[...], b1[...], True, p2)
    y = _conv3x3(y, w2[...], b2[...], False, p3)
    y = _conv3x3(y, w3[...], b3[...], True, p4)
    y = _conv3x3(y, w4[...], b4[...], False, p5)
    y = _conv3x3(y, w5[...], b5[...], True, p6)
    o_ref[...] = y.reshape(y.shape[0], 64, 128)


def _head_body(f_ref, w1_ref, b1_ref, w2_ref, b2_ref, o_ref):
    h = jnp.dot(f_ref[...], w1_ref[...], preferred_element_type=jnp.float32)
    h = jnp.maximum(h + b1_ref[...], 0.0)
    o_ref[...] = (jnp.dot(h, w2_ref[...], preferred_element_type=jnp.float32)
                  + b2_ref[...])


def kernel(x, cw0, cb0, cw1, cb1, cw2, cb2, cw3, cb3, cw4, cb4, cw5, cb5,
           fw0, fb0, fw1, fb1):
    B = x.shape[0]
    BT = 4
    xh = jnp.transpose(x, (0, 2, 1, 3))

    ws, bs = [], []
    for w, b in ((cw0, cb0), (cw1, cb1), (cw2, cb2),
                 (cw3, cb3), (cw4, cb4), (cw5, cb5)):
        cin, cout = w.shape[2], w.shape[3]
        ws.append(w.reshape(9 * cin, cout).astype(jnp.bfloat16))
        bs.append(b.reshape(1, cout))

    def _whole(shape):
        return pl.BlockSpec(shape, lambda i: (0,) * len(shape))

    in_specs = [pl.BlockSpec((BT, 64, 3, 64), lambda i: (i, 0, 0, 0))]
    args = [xh]
    for w9, b in zip(ws, bs):
        in_specs += [_whole(w9.shape), _whole(b.shape)]
        args += [w9, b]

    flat3 = pl.pallas_call(
        _trunk_body,
        out_shape=jax.ShapeDtypeStruct((B, 64, 128), jnp.bfloat16),
        grid=(B // BT,),
        in_specs=in_specs,
        out_specs=pl.BlockSpec((BT, 64, 128), lambda i: (i, 0, 0)),
        scratch_shapes=[
            pltpu.VMEM((BT, 66, 64, 9), jnp.bfloat16),
            pltpu.VMEM((BT, 66, 8, 8, 96), jnp.bfloat16),
            pltpu.VMEM((BT, 34, 4, 8, 96), jnp.bfloat16),
            pltpu.VMEM((BT, 34, 4, 8, 192), jnp.bfloat16),
            pltpu.VMEM((BT, 18, 2, 8, 192), jnp.bfloat16),
            pltpu.VMEM((BT, 18, 2, 8, 384), jnp.bfloat16),
        ],
        compiler_params=pltpu.CompilerParams(
            dimension_semantics=("parallel",), vmem_limit_bytes=_VMEM),
    )(*args)
    flat = flat3.reshape(B, 64 * 128)

    fw0p = (fw0.reshape(128, 8, 8, 512).transpose(1, 2, 0, 3)
            .reshape(8192, 512))

    TM = min(256, B)
    out = pl.pallas_call(
        _head_body,
        out_shape=jax.ShapeDtypeStruct((B, 10), jnp.float32),
        grid=(B // TM,),
        in_specs=[pl.BlockSpec((TM, 8192), lambda i: (i, 0)),
                  _whole((8192, 512)), _whole((1, 512)),
                  _whole((512, 10)), _whole((1, 10))],
        out_specs=pl.BlockSpec((TM, 10), lambda i: (i, 0)),
        compiler_params=pltpu.CompilerParams(
            dimension_semantics=("parallel",), vmem_limit_bytes=_VMEM),
    )(flat, fw0p, fb0.reshape(1, 512), fw1, fb1.reshape(1, 10))
    return out
